# initial kernel scaffold (unmeasured)
import jax
import jax.numpy as jnp
from jax import lax
from jax.experimental import pallas as pl
from jax.experimental.pallas import tpu as pltpu

N_DEV = 8
HQ = 8
DH = 128
DMODEL = HQ * DH
SQ = 256
SCALE = 0.08838834764831843

PACK_W = DMODEL + 128
M_OFF = DMODEL
L_OFF = DMODEL + HQ
N_ROUNDS = 3


def kernel(x, Wq, Wo, K_ext, V_ext):
    skv = K_ext.shape[1]
    x2 = x.reshape(SQ, DMODEL)
    k2 = K_ext.reshape(skv, HQ, DH)
    v2 = V_ext.reshape(skv, HQ, DH)

    def body(x_ref, wq_ref, wo_ref, k_ref, v_ref, out_ref,
             pack_ref, recv_ref, send_sems, recv_sems):
        my = lax.axis_index("i")

        for h in range(HQ):
            qh = jnp.dot(x_ref[:, :], wq_ref[:, h * DH:(h + 1) * DH],
                         preferred_element_type=jnp.float32)
            s = lax.dot_general(qh, k_ref[:, h, :], (((1,), (1,)), ((), ())),
                                preferred_element_type=jnp.float32) * SCALE
            m = jnp.max(s, axis=1, keepdims=True)
            p = jnp.exp(s - m)
            l = jnp.sum(p, axis=1, keepdims=True)
            o = jnp.dot(p, v_ref[:, h, :], preferred_element_type=jnp.float32)
            pack_ref[:, h * DH:(h + 1) * DH] = o
            pack_ref[:, M_OFF + h:M_OFF + h + 1] = m
            pack_ref[:, L_OFF + h:L_OFF + h + 1] = l

        for r in range(N_ROUNDS):
            partner = my ^ (1 << r)
            rdma = pltpu.make_async_remote_copy(
                src_ref=pack_ref,
                dst_ref=recv_ref.at[r],
                send_sem=send_sems.at[r],
                recv_sem=recv_sems.at[r],
                device_id=(partner,),
                device_id_type=pl.DeviceIdType.MESH,
            )
            rdma.start()
            rdma.wait()

            m_a = pack_ref[:, M_OFF:M_OFF + HQ]
            l_a = pack_ref[:, L_OFF:L_OFF + HQ]
            m_b = recv_ref[r, :, M_OFF:M_OFF + HQ]
            l_b = recv_ref[r, :, L_OFF:L_OFF + HQ]
            m_n = jnp.maximum(m_a, m_b)
            a = jnp.exp(m_a - m_n)
            b = jnp.exp(m_b - m_n)
            for h in range(HQ):
                pack_ref[:, h * DH:(h + 1) * DH] = (
                    pack_ref[:, h * DH:(h + 1) * DH] * a[:, h:h + 1]
                    + recv_ref[r, :, h * DH:(h + 1) * DH] * b[:, h:h + 1])
            pack_ref[:, M_OFF:M_OFF + HQ] = m_n
            pack_ref[:, L_OFF:L_OFF + HQ] = l_a * a + l_b * b

        l_fin = pack_ref[:, L_OFF:L_OFF + HQ]
        for h in range(HQ):
            pack_ref[:, h * DH:(h + 1) * DH] = (
                pack_ref[:, h * DH:(h + 1) * DH] / l_fin[:, h:h + 1])
        out_ref[:, :] = jnp.dot(pack_ref[:, :DMODEL], wo_ref[:, :],
                                preferred_element_type=jnp.float32)

    out = pl.pallas_call(
        body,
        out_shape=jax.ShapeDtypeStruct((SQ, DMODEL), jnp.float32),
        in_specs=[pl.BlockSpec(memory_space=pltpu.VMEM)] * 5,
        out_specs=pl.BlockSpec(memory_space=pltpu.VMEM),
        scratch_shapes=[
            pltpu.VMEM((SQ, PACK_W), jnp.float32),
            pltpu.VMEM((N_ROUNDS, SQ, PACK_W), jnp.float32),
            pltpu.SemaphoreType.DMA((N_ROUNDS,)),
            pltpu.SemaphoreType.DMA((N_ROUNDS,)),
        ],
        compiler_params=pltpu.CompilerParams(collective_id=0),
    )(x2, Wq, Wo, k2, v2)
    return out.reshape(1, SQ, DMODEL)


# baseline (device time: 72033 ns/iter reference)
import jax
import jax.numpy as jnp
from jax import lax
from jax.experimental import pallas as pl
from jax.experimental.pallas import tpu as pltpu

N_DEV = 8
HQ = 8
DH = 128
DMODEL = HQ * DH
SQ = 256
SCALE = 0.08838834764831843

PACK_W = DMODEL + 128
M_OFF = DMODEL
L_OFF = DMODEL + HQ
N_ROUNDS = 3


def kernel(x, Wq, Wo, K_ext, V_ext):
    skv = K_ext.shape[1]
    x2 = x.reshape(SQ, DMODEL)
    k2 = K_ext.reshape(skv, HQ, DH)
    v2 = V_ext.reshape(skv, HQ, DH)

    def body(x_ref, wq_ref, wo_ref, k_ref, v_ref, out_ref,
             kbuf, vbuf, acc_ref, stat_ref, pack_ref, recv_ref,
             kv_sems, send_sems, recv_sems):
        my = lax.axis_index("i")

        def fetch(h, slot):
            pltpu.make_async_copy(
                k_ref.at[:, h, :], kbuf.at[slot], kv_sems.at[slot, 0]
            ).start()
            pltpu.make_async_copy(
                v_ref.at[:, h, :], vbuf.at[slot], kv_sems.at[slot, 1]
            ).start()

        def fetch_wait(h, slot):
            pltpu.make_async_copy(
                k_ref.at[:, h, :], kbuf.at[slot], kv_sems.at[slot, 0]
            ).wait()
            pltpu.make_async_copy(
                v_ref.at[:, h, :], vbuf.at[slot], kv_sems.at[slot, 1]
            ).wait()

        fetch(0, 0)
        for h in range(HQ):
            slot = h % 2
            if h + 1 < HQ:
                fetch(h + 1, (h + 1) % 2)
            fetch_wait(h, slot)
            qh = jnp.dot(x_ref[:, :], wq_ref[:, h * DH:(h + 1) * DH],
                         preferred_element_type=jnp.float32)
            s = lax.dot_general(qh, kbuf[slot], (((1,), (1,)), ((), ())),
                                preferred_element_type=jnp.float32) * SCALE
            m = jnp.max(s, axis=1, keepdims=True)
            p = jnp.exp(s - m)
            l = jnp.sum(p, axis=1, keepdims=True)
            o = jnp.dot(p, vbuf[slot], preferred_element_type=jnp.float32)
            acc_ref[:, h * DH:(h + 1) * DH] = o
            stat_ref[:, h:h + 1] = m
            stat_ref[:, HQ + h:HQ + h + 1] = l

        for r in range(N_ROUNDS):
            pack_ref[:, :DMODEL] = acc_ref[:, :].astype(jnp.bfloat16)
            pack_ref[:, M_OFF:M_OFF + 2 * HQ] = (
                stat_ref[:, :2 * HQ].astype(jnp.bfloat16))

            partner = my ^ (1 << r)
            rdma = pltpu.make_async_remote_copy(
                src_ref=pack_ref,
                dst_ref=recv_ref.at[r],
                send_sem=send_sems.at[r],
                recv_sem=recv_sems.at[r],
                device_id=(partner,),
                device_id_type=pl.DeviceIdType.MESH,
            )
            rdma.start()
            rdma.wait()

            m_a = stat_ref[:, 0:HQ]
            l_a = stat_ref[:, HQ:2 * HQ]
            m_b = recv_ref[r, :, M_OFF:M_OFF + HQ].astype(jnp.float32)
            l_b = recv_ref[r, :, L_OFF:L_OFF + HQ].astype(jnp.float32)
            m_n = jnp.maximum(m_a, m_b)
            a = jnp.exp(m_a - m_n)
            b = jnp.exp(m_b - m_n)
            for h in range(HQ):
                acc_ref[:, h * DH:(h + 1) * DH] = (
                    acc_ref[:, h * DH:(h + 1) * DH] * a[:, h:h + 1]
                    + recv_ref[r, :, h * DH:(h + 1) * DH].astype(jnp.float32)
                    * b[:, h:h + 1])
            stat_ref[:, 0:HQ] = m_n
            stat_ref[:, HQ:2 * HQ] = l_a * a + l_b * b

        l_fin = stat_ref[:, HQ:2 * HQ]
        for h in range(HQ):
            acc_ref[:, h * DH:(h + 1) * DH] = (
                acc_ref[:, h * DH:(h + 1) * DH] / l_fin[:, h:h + 1])
        out_ref[:, :] = jnp.dot(acc_ref[:, :], wo_ref[:, :],
                                preferred_element_type=jnp.float32)

    out = pl.pallas_call(
        body,
        out_shape=jax.ShapeDtypeStruct((SQ, DMODEL), jnp.float32),
        in_specs=[
            pl.BlockSpec(memory_space=pltpu.VMEM),
            pl.BlockSpec(memory_space=pltpu.VMEM),
            pl.BlockSpec(memory_space=pltpu.VMEM),
            pl.BlockSpec(memory_space=pl.ANY),
            pl.BlockSpec(memory_space=pl.ANY),
        ],
        out_specs=pl.BlockSpec(memory_space=pltpu.VMEM),
        scratch_shapes=[
            pltpu.VMEM((2, skv, DH), jnp.float32),
            pltpu.VMEM((2, skv, DH), jnp.float32),
            pltpu.VMEM((SQ, DMODEL), jnp.float32),
            pltpu.VMEM((SQ, 128), jnp.float32),
            pltpu.VMEM((SQ, PACK_W), jnp.bfloat16),
            pltpu.VMEM((N_ROUNDS, SQ, PACK_W), jnp.bfloat16),
            pltpu.SemaphoreType.DMA((2, 2)),
            pltpu.SemaphoreType.DMA((N_ROUNDS,)),
            pltpu.SemaphoreType.DMA((N_ROUNDS,)),
        ],
        compiler_params=pltpu.CompilerParams(
            vmem_limit_bytes=100 * 1024 * 1024,
        ),
    )(x2, Wq, Wo, k2, v2)
    return out.reshape(1, SQ, DMODEL)


# device time: 63936 ns/iter; 1.1266x vs baseline; 1.1266x over previous
import jax
import jax.numpy as jnp
from jax import lax
from jax.experimental import pallas as pl
from jax.experimental.pallas import tpu as pltpu

N_DEV = 8
HQ = 8
DH = 128
DMODEL = HQ * DH
SQ = 256
SCALE = 0.08838834764831843

N_GRP = 2
HPG = HQ // N_GRP
GW = HPG * DH
PACK_W = GW + 128
M_OFF = GW
L_OFF = GW + HPG
N_ROUNDS = 3


def kernel(x, Wq, Wo, K_ext, V_ext):
    skv = K_ext.shape[1]
    x2 = x.reshape(SQ, DMODEL)
    k2 = K_ext.reshape(skv, HQ, DH)
    v2 = V_ext.reshape(skv, HQ, DH)

    def body(x_ref, wq_ref, wo_ref, k_ref, v_ref, out_ref,
             kbuf, vbuf, q_ref, acc_ref, stat_ref, pack_ref, recv_ref,
             kv_sems, send_sems, recv_sems):
        my = lax.axis_index("i")

        def fetch(h, slot):
            pltpu.make_async_copy(
                k_ref.at[:, h, :], kbuf.at[slot], kv_sems.at[slot, 0]
            ).start()
            pltpu.make_async_copy(
                v_ref.at[:, h, :], vbuf.at[slot], kv_sems.at[slot, 1]
            ).start()

        def fetch_wait(h, slot):
            pltpu.make_async_copy(
                k_ref.at[:, h, :], kbuf.at[slot], kv_sems.at[slot, 0]
            ).wait()
            pltpu.make_async_copy(
                v_ref.at[:, h, :], vbuf.at[slot], kv_sems.at[slot, 1]
            ).wait()

        def grp_rdma(g, r):
            return pltpu.make_async_remote_copy(
                src_ref=pack_ref.at[g],
                dst_ref=recv_ref.at[g, r],
                send_sem=send_sems.at[g, r],
                recv_sem=recv_sems.at[g, r],
                device_id=(my ^ (1 << r),),
                device_id_type=pl.DeviceIdType.MESH,
            )

        def cast_and_send(g, r):
            pack_ref[g, :, :GW] = (
                acc_ref[:, g * GW:(g + 1) * GW].astype(jnp.bfloat16))
            pack_ref[g, :, M_OFF:M_OFF + HPG] = (
                stat_ref[:, g * HPG:(g + 1) * HPG].astype(jnp.bfloat16))
            pack_ref[g, :, L_OFF:L_OFF + HPG] = (
                stat_ref[:, HQ + g * HPG:HQ + (g + 1) * HPG]
                .astype(jnp.bfloat16))
            grp_rdma(g, r).start()

        def combine(g, r):
            m_a = stat_ref[:, g * HPG:(g + 1) * HPG]
            l_a = stat_ref[:, HQ + g * HPG:HQ + (g + 1) * HPG]
            m_b = recv_ref[g, r, :, M_OFF:M_OFF + HPG].astype(jnp.float32)
            l_b = recv_ref[g, r, :, L_OFF:L_OFF + HPG].astype(jnp.float32)
            m_n = jnp.maximum(m_a, m_b)
            a = jnp.exp(m_a - m_n)
            b = jnp.exp(m_b - m_n)
            for i in range(HPG):
                h = g * HPG + i
                acc_ref[:, h * DH:(h + 1) * DH] = (
                    acc_ref[:, h * DH:(h + 1) * DH] * a[:, i:i + 1]
                    + recv_ref[g, r, :, i * DH:(i + 1) * DH]
                    .astype(jnp.float32) * b[:, i:i + 1])
            stat_ref[:, g * HPG:(g + 1) * HPG] = m_n
            stat_ref[:, HQ + g * HPG:HQ + (g + 1) * HPG] = l_a * a + l_b * b

        fetch(0, 0)
        q_ref[:, :] = jnp.dot(x_ref[:, :], wq_ref[:, :],
                              preferred_element_type=jnp.float32)
        for h in range(HQ):
            slot = h % 2
            if h + 1 < HQ:
                fetch(h + 1, (h + 1) % 2)
            fetch_wait(h, slot)
            qh = q_ref[:, h * DH:(h + 1) * DH]
            s = lax.dot_general(qh, kbuf[slot], (((1,), (1,)), ((), ())),
                                preferred_element_type=jnp.float32) * SCALE
            m = jnp.max(s, axis=1, keepdims=True)
            p = jnp.exp(s - m)
            l = jnp.sum(p, axis=1, keepdims=True)
            o = jnp.dot(p, vbuf[slot], preferred_element_type=jnp.float32)
            acc_ref[:, h * DH:(h + 1) * DH] = o
            stat_ref[:, h:h + 1] = m
            stat_ref[:, HQ + h:HQ + h + 1] = l
            if h == HPG - 1:
                cast_and_send(0, 0)

        cast_and_send(1, 0)

        for r in range(N_ROUNDS):
            for g in range(N_GRP):
                grp_rdma(g, r).wait()
                combine(g, r)
                if r + 1 < N_ROUNDS:
                    cast_and_send(g, r + 1)

        for h in range(HQ):
            acc_ref[:, h * DH:(h + 1) * DH] = (
                acc_ref[:, h * DH:(h + 1) * DH]
                / stat_ref[:, HQ + h:HQ + h + 1])
        out_ref[:, :] = jnp.dot(acc_ref[:, :], wo_ref[:, :],
                                preferred_element_type=jnp.float32)

    out = pl.pallas_call(
        body,
        out_shape=jax.ShapeDtypeStruct((SQ, DMODEL), jnp.float32),
        in_specs=[
            pl.BlockSpec(memory_space=pltpu.VMEM),
            pl.BlockSpec(memory_space=pltpu.VMEM),
            pl.BlockSpec(memory_space=pltpu.VMEM),
            pl.BlockSpec(memory_space=pl.ANY),
            pl.BlockSpec(memory_space=pl.ANY),
        ],
        out_specs=pl.BlockSpec(memory_space=pltpu.VMEM),
        scratch_shapes=[
            pltpu.VMEM((2, skv, DH), jnp.float32),
            pltpu.VMEM((2, skv, DH), jnp.float32),
            pltpu.VMEM((SQ, DMODEL), jnp.float32),
            pltpu.VMEM((SQ, DMODEL), jnp.float32),
            pltpu.VMEM((SQ, 128), jnp.float32),
            pltpu.VMEM((N_GRP, SQ, PACK_W), jnp.bfloat16),
            pltpu.VMEM((N_GRP, N_ROUNDS, SQ, PACK_W), jnp.bfloat16),
            pltpu.SemaphoreType.DMA((2, 2)),
            pltpu.SemaphoreType.DMA((N_GRP, N_ROUNDS)),
            pltpu.SemaphoreType.DMA((N_GRP, N_ROUNDS)),
        ],
        compiler_params=pltpu.CompilerParams(
            vmem_limit_bytes=100 * 1024 * 1024,
        ),
    )(x2, Wq, Wo, k2, v2)
    return out.reshape(1, SQ, DMODEL)


# device time: 51847 ns/iter; 1.3893x vs baseline; 1.2332x over previous
import jax
import jax.numpy as jnp
from jax import lax
from jax.experimental import pallas as pl
from jax.experimental.pallas import tpu as pltpu

N_DEV = 8
HQ = 8
DH = 128
DMODEL = HQ * DH
SQ = 256
SCALE = 0.08838834764831843

N_GRP = 2
HPG = HQ // N_GRP
GW = HPG * DH
L_OFF = GW
PAYLOAD_W = GW + HPG
PACK_W = GW + 128
N_ROUNDS = 3


def kernel(x, Wq, Wo, K_ext, V_ext):
    skv = K_ext.shape[1]
    x2 = x.reshape(SQ, DMODEL)
    k2 = K_ext.reshape(skv, HQ, DH)
    v2 = V_ext.reshape(skv, HQ, DH)

    def body(x_ref, wq_ref, wo_ref, k_ref, v_ref, out_ref,
             kbuf, vbuf, q_ref, pack_ref, recv_ref,
             kv_sems, send_sems, recv_sems):
        my = lax.axis_index("i")

        def fetch(h, slot):
            pltpu.make_async_copy(
                k_ref.at[:, h, :], kbuf.at[slot], kv_sems.at[slot, 0]
            ).start()
            pltpu.make_async_copy(
                v_ref.at[:, h, :], vbuf.at[slot], kv_sems.at[slot, 1]
            ).start()

        def fetch_wait(h, slot):
            pltpu.make_async_copy(
                k_ref.at[:, h, :], kbuf.at[slot], kv_sems.at[slot, 0]
            ).wait()
            pltpu.make_async_copy(
                v_ref.at[:, h, :], vbuf.at[slot], kv_sems.at[slot, 1]
            ).wait()

        def grp_rdma(g, r):
            return pltpu.make_async_remote_copy(
                src_ref=pack_ref.at[g],
                dst_ref=recv_ref.at[g, r],
                send_sem=send_sems.at[g, r],
                recv_sem=recv_sems.at[g, r],
                device_id=(my ^ (1 << r),),
                device_id_type=pl.DeviceIdType.MESH,
            )

        fetch(0, 0)
        q_ref[:, :] = jnp.dot(x_ref[:, :], wq_ref[:, :],
                              preferred_element_type=jnp.float32) * SCALE
        for h in range(HQ):
            slot = h % 2
            if h + 1 < HQ:
                fetch(h + 1, (h + 1) % 2)
            fetch_wait(h, slot)
            qh = q_ref[:, h * DH:(h + 1) * DH]
            s = lax.dot_general(qh, kbuf[slot], (((1,), (1,)), ((), ())),
                                preferred_element_type=jnp.float32)
            p = jnp.exp(s)
            l = jnp.sum(p, axis=1, keepdims=True)
            o = jnp.dot(p, vbuf[slot], preferred_element_type=jnp.float32)
            g, i = divmod(h, HPG)
            pack_ref[g, :, i * DH:(i + 1) * DH] = o.astype(jnp.bfloat16)
            pack_ref[g, :, L_OFF + i:L_OFF + i + 1] = l.astype(jnp.bfloat16)
            if h == HPG - 1:
                grp_rdma(0, 0).start()

        grp_rdma(1, 0).start()

        for r in range(N_ROUNDS):
            for g in range(N_GRP):
                grp_rdma(g, r).wait()
                pack_ref[g, :, :PAYLOAD_W] = (
                    pack_ref[g, :, :PAYLOAD_W]
                    + recv_ref[g, r, :, :PAYLOAD_W])
                if r + 1 < N_ROUNDS:
                    grp_rdma(g, r + 1).start()

        for h in range(HQ):
            g, i = divmod(h, HPG)
            q_ref[:, h * DH:(h + 1) * DH] = (
                pack_ref[g, :, i * DH:(i + 1) * DH].astype(jnp.float32)
                / pack_ref[g, :, L_OFF + i:L_OFF + i + 1]
                .astype(jnp.float32))
        out_ref[:, :] = jnp.dot(q_ref[:, :], wo_ref[:, :],
                                preferred_element_type=jnp.float32)

    out = pl.pallas_call(
        body,
        out_shape=jax.ShapeDtypeStruct((SQ, DMODEL), jnp.float32),
        in_specs=[
            pl.BlockSpec(memory_space=pltpu.VMEM),
            pl.BlockSpec(memory_space=pltpu.VMEM),
            pl.BlockSpec(memory_space=pltpu.VMEM),
            pl.BlockSpec(memory_space=pl.ANY),
            pl.BlockSpec(memory_space=pl.ANY),
        ],
        out_specs=pl.BlockSpec(memory_space=pltpu.VMEM),
        scratch_shapes=[
            pltpu.VMEM((2, skv, DH), jnp.float32),
            pltpu.VMEM((2, skv, DH), jnp.float32),
            pltpu.VMEM((SQ, DMODEL), jnp.float32),
            pltpu.VMEM((N_GRP, SQ, PACK_W), jnp.bfloat16),
            pltpu.VMEM((N_GRP, N_ROUNDS, SQ, PACK_W), jnp.bfloat16),
            pltpu.SemaphoreType.DMA((2, 2)),
            pltpu.SemaphoreType.DMA((N_GRP, N_ROUNDS)),
            pltpu.SemaphoreType.DMA((N_GRP, N_ROUNDS)),
        ],
        compiler_params=pltpu.CompilerParams(
            vmem_limit_bytes=100 * 1024 * 1024,
        ),
    )(x2, Wq, Wo, k2, v2)
    return out.reshape(1, SQ, DMODEL)


# device time: 50564 ns/iter; 1.4246x vs baseline; 1.0254x over previous
import jax
import jax.numpy as jnp
from jax import lax
from jax.experimental import pallas as pl
from jax.experimental.pallas import tpu as pltpu

N_DEV = 8
HQ = 8
DH = 128
DMODEL = HQ * DH
SQ = 256
SCALE = 0.08838834764831843

N_GRP = 4
HPG = HQ // N_GRP
GW = HPG * DH
L_OFF = GW
PAYLOAD_W = GW + HPG
PACK_W = GW + 128
N_ROUNDS = 3


def kernel(x, Wq, Wo, K_ext, V_ext):
    skv = K_ext.shape[1]
    x2 = x.reshape(SQ, DMODEL)
    k2 = K_ext.reshape(skv, HQ, DH)
    v2 = V_ext.reshape(skv, HQ, DH)

    def body(x_ref, wq_ref, wo_ref, k_ref, v_ref, out_ref,
             kbuf, vbuf, q_ref, pack_ref, recv_ref,
             kv_sems, send_sems, recv_sems):
        my = lax.axis_index("i")

        def fetch(h, slot):
            pltpu.make_async_copy(
                k_ref.at[:, h, :], kbuf.at[slot], kv_sems.at[slot, 0]
            ).start()
            pltpu.make_async_copy(
                v_ref.at[:, h, :], vbuf.at[slot], kv_sems.at[slot, 1]
            ).start()

        def fetch_wait(h, slot):
            pltpu.make_async_copy(
                k_ref.at[:, h, :], kbuf.at[slot], kv_sems.at[slot, 0]
            ).wait()
            pltpu.make_async_copy(
                v_ref.at[:, h, :], vbuf.at[slot], kv_sems.at[slot, 1]
            ).wait()

        def grp_rdma(g, r):
            return pltpu.make_async_remote_copy(
                src_ref=pack_ref.at[g],
                dst_ref=recv_ref.at[g, r],
                send_sem=send_sems.at[g, r],
                recv_sem=recv_sems.at[g, r],
                device_id=(my ^ (1 << r),),
                device_id_type=pl.DeviceIdType.MESH,
            )

        fetch(0, 0)
        q_ref[:, :] = jnp.dot(x_ref[:, :], wq_ref[:, :],
                              preferred_element_type=jnp.float32) * SCALE
        for h in range(HQ):
            slot = h % 2
            if h + 1 < HQ:
                fetch(h + 1, (h + 1) % 2)
            fetch_wait(h, slot)
            qh = q_ref[:, h * DH:(h + 1) * DH]
            s = lax.dot_general(qh, kbuf[slot], (((1,), (1,)), ((), ())),
                                preferred_element_type=jnp.float32)
            p = jnp.exp(s)
            l = jnp.sum(p, axis=1, keepdims=True)
            o = jnp.dot(p, vbuf[slot], preferred_element_type=jnp.float32)
            g, i = divmod(h, HPG)
            pack_ref[g, :, i * DH:(i + 1) * DH] = o.astype(jnp.bfloat16)
            pack_ref[g, :, L_OFF + i:L_OFF + i + 1] = l.astype(jnp.bfloat16)
            if i == HPG - 1:
                grp_rdma(g, 0).start()

        for r in range(N_ROUNDS):
            for g in range(N_GRP):
                grp_rdma(g, r).wait()
                pack_ref[g, :, :PAYLOAD_W] = (
                    pack_ref[g, :, :PAYLOAD_W]
                    + recv_ref[g, r, :, :PAYLOAD_W])
                if r + 1 < N_ROUNDS:
                    grp_rdma(g, r + 1).start()

        for h in range(HQ):
            g, i = divmod(h, HPG)
            q_ref[:, h * DH:(h + 1) * DH] = (
                pack_ref[g, :, i * DH:(i + 1) * DH].astype(jnp.float32)
                / pack_ref[g, :, L_OFF + i:L_OFF + i + 1]
                .astype(jnp.float32))
        out_ref[:, :] = jnp.dot(q_ref[:, :], wo_ref[:, :],
                                preferred_element_type=jnp.float32)

    out = pl.pallas_call(
        body,
        out_shape=jax.ShapeDtypeStruct((SQ, DMODEL), jnp.float32),
        in_specs=[
            pl.BlockSpec(memory_space=pltpu.VMEM),
            pl.BlockSpec(memory_space=pltpu.VMEM),
            pl.BlockSpec(memory_space=pltpu.VMEM),
            pl.BlockSpec(memory_space=pl.ANY),
            pl.BlockSpec(memory_space=pl.ANY),
        ],
        out_specs=pl.BlockSpec(memory_space=pltpu.VMEM),
        scratch_shapes=[
            pltpu.VMEM((2, skv, DH), jnp.float32),
            pltpu.VMEM((2, skv, DH), jnp.float32),
            pltpu.VMEM((SQ, DMODEL), jnp.float32),
            pltpu.VMEM((N_GRP, SQ, PACK_W), jnp.bfloat16),
            pltpu.VMEM((N_GRP, N_ROUNDS, SQ, PACK_W), jnp.bfloat16),
            pltpu.SemaphoreType.DMA((2, 2)),
            pltpu.SemaphoreType.DMA((N_GRP, N_ROUNDS)),
            pltpu.SemaphoreType.DMA((N_GRP, N_ROUNDS)),
        ],
        compiler_params=pltpu.CompilerParams(
            vmem_limit_bytes=100 * 1024 * 1024,
        ),
    )(x2, Wq, Wo, k2, v2)
    return out.reshape(1, SQ, DMODEL)


# device time: 46596 ns/iter; 1.5459x vs baseline; 1.0852x over previous
import jax
import jax.numpy as jnp
from jax import lax
from jax.experimental import pallas as pl
from jax.experimental.pallas import tpu as pltpu

N_DEV = 8
HQ = 8
DH = 128
DMODEL = HQ * DH
SQ = 256
SCALE = 0.08838834764831843

N_GRP = 4
HPG = HQ // N_GRP
GW = HPG * DH
L_OFF = GW
PAYLOAD_W = GW + HPG
PACK_W = GW + 128
N_ROUNDS = 3


def kernel(x, Wq, Wo, K_ext, V_ext):
    skv = K_ext.shape[1]
    x2 = x.reshape(SQ, DMODEL)
    k2 = K_ext.reshape(skv, HQ, DH)
    v2 = V_ext.reshape(skv, HQ, DH)

    def body(x_ref, wq_ref, wo_ref, k_ref, v_ref, out_ref,
             kbuf, vbuf, q_ref, pack_ref, recv_ref,
             kv_sems, send_sems, recv_sems):
        my = lax.axis_index("i")

        def fetch(h, slot):
            pltpu.make_async_copy(
                k_ref.at[:, h, :], kbuf.at[slot], kv_sems.at[slot, 0]
            ).start()
            pltpu.make_async_copy(
                v_ref.at[:, h, :], vbuf.at[slot], kv_sems.at[slot, 1]
            ).start()

        def fetch_wait(h, slot):
            pltpu.make_async_copy(
                k_ref.at[:, h, :], kbuf.at[slot], kv_sems.at[slot, 0]
            ).wait()
            pltpu.make_async_copy(
                v_ref.at[:, h, :], vbuf.at[slot], kv_sems.at[slot, 1]
            ).wait()

        def grp_rdma(g, r):
            return pltpu.make_async_remote_copy(
                src_ref=pack_ref.at[g],
                dst_ref=recv_ref.at[g, r],
                send_sem=send_sems.at[g, r],
                recv_sem=recv_sems.at[g, r],
                device_id=(my ^ (1 << r),),
                device_id_type=pl.DeviceIdType.MESH,
            )

        barrier_sem = pltpu.get_barrier_semaphore()
        for r in range(N_ROUNDS):
            pl.semaphore_signal(barrier_sem, inc=1,
                                device_id=(my ^ (1 << r),),
                                device_id_type=pl.DeviceIdType.MESH)
        pl.semaphore_wait(barrier_sem, N_ROUNDS)

        fetch(0, 0)
        q_ref[:, :] = jnp.dot(x_ref[:, :], wq_ref[:, :],
                              preferred_element_type=jnp.float32) * SCALE
        for h in range(HQ):
            slot = h % 2
            if h + 1 < HQ:
                fetch(h + 1, (h + 1) % 2)
            fetch_wait(h, slot)
            qh = q_ref[:, h * DH:(h + 1) * DH]
            s = lax.dot_general(qh, kbuf[slot], (((1,), (1,)), ((), ())),
                                preferred_element_type=jnp.float32)
            p = jnp.exp(s)
            l = jnp.sum(p, axis=1, keepdims=True)
            o = jnp.dot(p, vbuf[slot], preferred_element_type=jnp.float32)
            g, i = divmod(h, HPG)
            pack_ref[g, :, i * DH:(i + 1) * DH] = o.astype(jnp.bfloat16)
            pack_ref[g, :, L_OFF + i:L_OFF + i + 1] = l.astype(jnp.bfloat16)
            if i == HPG - 1:
                grp_rdma(g, 0).start()

        for r in range(N_ROUNDS):
            for g in range(N_GRP):
                grp_rdma(g, r).wait()
                pack_ref[g, :, :PAYLOAD_W] = (
                    pack_ref[g, :, :PAYLOAD_W]
                    + recv_ref[g, r, :, :PAYLOAD_W])
                if r + 1 < N_ROUNDS:
                    grp_rdma(g, r + 1).start()

        for h in range(HQ):
            g, i = divmod(h, HPG)
            q_ref[:, h * DH:(h + 1) * DH] = (
                pack_ref[g, :, i * DH:(i + 1) * DH].astype(jnp.float32)
                / pack_ref[g, :, L_OFF + i:L_OFF + i + 1]
                .astype(jnp.float32))
        out_ref[:, :] = jnp.dot(q_ref[:, :], wo_ref[:, :],
                                preferred_element_type=jnp.float32)

    out = pl.pallas_call(
        body,
        out_shape=jax.ShapeDtypeStruct((SQ, DMODEL), jnp.float32),
        in_specs=[
            pl.BlockSpec(memory_space=pltpu.VMEM),
            pl.BlockSpec(memory_space=pltpu.VMEM),
            pl.BlockSpec(memory_space=pltpu.VMEM),
            pl.BlockSpec(memory_space=pl.ANY),
            pl.BlockSpec(memory_space=pl.ANY),
        ],
        out_specs=pl.BlockSpec(memory_space=pltpu.VMEM),
        scratch_shapes=[
            pltpu.VMEM((2, skv, DH), jnp.float32),
            pltpu.VMEM((2, skv, DH), jnp.float32),
            pltpu.VMEM((SQ, DMODEL), jnp.float32),
            pltpu.VMEM((N_GRP, SQ, PACK_W), jnp.bfloat16),
            pltpu.VMEM((N_GRP, N_ROUNDS, SQ, PACK_W), jnp.bfloat16),
            pltpu.SemaphoreType.DMA((2, 2)),
            pltpu.SemaphoreType.DMA((N_GRP, N_ROUNDS)),
            pltpu.SemaphoreType.DMA((N_GRP, N_ROUNDS)),
        ],
        compiler_params=pltpu.CompilerParams(
            vmem_limit_bytes=100 * 1024 * 1024,
            collective_id=0,
        ),
    )(x2, Wq, Wo, k2, v2)
    return out.reshape(1, SQ, DMODEL)
